# baseline (speedup 1.0000x reference)
import functools

import jax
import jax.numpy as jnp
from jax import lax
from jax.experimental import pallas as pl
from jax.experimental.pallas import tpu as pltpu
from jax.experimental.pallas import tpu_sc as plsc

N, K, D = 16384, 100000, 64
NC, NS, LANES = 2, 16, 16
NW = NC * NS
ROWS_PER_W = N // NW
CHUNK = 128
NCHUNK = ROWS_PER_W // CHUNK
PCHUNK = 16384
NSTEP = -(-K // PCHUNK)
KP = NSTEP * PCHUNK


def _pack_body(eT_ref, vT_ref, ev_ref):
    i = lax.broadcasted_iota(jnp.int32, (D, D), 0)
    j = lax.broadcasted_iota(jnp.int32, (D, D), 1)
    eye = (i == j).astype(jnp.float32)
    dims = (((0,), (0,)), ((), ()))
    ev_ref[:, 0:D] = eT_ref[...].T
    ev_ref[:, D:2 * D] = lax.dot_general(
        vT_ref[...], eye, dims, preferred_element_type=jnp.float32)


def _pack_tables(exT, varT):
    return pl.pallas_call(
        _pack_body,
        grid=(NSTEP,),
        in_specs=[
            pl.BlockSpec((D, PCHUNK), lambda i: (0, i)),
            pl.BlockSpec((D, PCHUNK), lambda i: (0, i)),
        ],
        out_specs=pl.BlockSpec((PCHUNK, 2 * D), lambda i: (i, 0)),
        out_shape=jax.ShapeDtypeStruct((KP, 2 * D), jnp.float32),
    )(exT, varT)


def _sc_body(featT_hbm, idx_hbm, ev_hbm, out_hbm,
             idx_v, feat_v, ev_v, acc_v, sems):
    c = lax.axis_index("c")
    s = lax.axis_index("s")
    wid = s * NC + c
    base = wid * ROWS_PER_W

    pltpu.sync_copy(idx_hbm.at[pl.ds(base, ROWS_PER_W)], idx_v)
    pltpu.sync_copy(featT_hbm.at[:, pl.ds(base, ROWS_PER_W)], feat_v)

    def gather(j, buf):
        isl = idx_v.at[pl.ds(j * CHUNK, CHUNK)]
        return pltpu.async_copy(ev_hbm.at[isl], ev_v.at[buf], sems.at[buf])

    iota = lax.iota(jnp.int32, LANES)
    zero = jnp.zeros((LANES,), jnp.float32)
    accs = (zero, zero, zero, zero)
    pending = gather(0, 0)
    for j in range(NCHUNK):
        pending.wait()
        if j + 1 < NCHUNK:
            pending = gather(j + 1, (j + 1) % 2)
        buf = j % 2

        def row_body(r, accs):
            a0, a1, a2, a3 = accs
            p = jnp.full((LANES,), j * CHUNK + r, jnp.int32)
            f0 = plsc.load_gather(feat_v, [iota, p])
            e0 = ev_v[buf, r, pl.ds(0, LANES)]
            v0 = ev_v[buf, r, pl.ds(D, LANES)]
            a0 = a0 + jnp.abs(f0 - e0) / v0
            f1 = plsc.load_gather(feat_v, [iota + LANES, p])
            e1 = ev_v[buf, r, pl.ds(LANES, LANES)]
            v1 = ev_v[buf, r, pl.ds(D + LANES, LANES)]
            a1 = a1 + jnp.abs(f1 - e1) / v1
            f2 = plsc.load_gather(feat_v, [iota + 2 * LANES, p])
            e2 = ev_v[buf, r, pl.ds(2 * LANES, LANES)]
            v2 = ev_v[buf, r, pl.ds(D + 2 * LANES, LANES)]
            a2 = a2 + jnp.abs(f2 - e2) / v2
            f3 = plsc.load_gather(feat_v, [iota + 3 * LANES, p])
            e3 = ev_v[buf, r, pl.ds(3 * LANES, LANES)]
            v3 = ev_v[buf, r, pl.ds(D + 3 * LANES, LANES)]
            a3 = a3 + jnp.abs(f3 - e3) / v3
            return (a0, a1, a2, a3)

        accs = lax.fori_loop(0, CHUNK, row_body, accs)

    acc_v[...] = (accs[0] + accs[1]) + (accs[2] + accs[3])
    pltpu.sync_copy(acc_v, out_hbm.at[wid])


@jax.jit
def _sc_loss(featT, idx, exT, varT):
    ev2 = _pack_tables(exT, varT)
    mesh = plsc.VectorSubcoreMesh(core_axis_name="c", subcore_axis_name="s")
    partials = pl.kernel(
        _sc_body,
        mesh=mesh,
        out_type=jax.ShapeDtypeStruct((NW, LANES), jnp.float32),
        compiler_params=pltpu.CompilerParams(
            use_tc_tiling_on_sc=True, needs_layout_passes=False),
        scratch_types=[
            pltpu.VMEM((ROWS_PER_W,), jnp.int32),
            pltpu.VMEM((D, ROWS_PER_W), jnp.float32),
            pltpu.VMEM((2, CHUNK, 2 * D), jnp.float32),
            pltpu.VMEM((LANES,), jnp.float32),
            pltpu.SemaphoreType.DMA((2,)),
        ],
    )(featT, idx, ev2)
    return jnp.sum(partials) / jnp.float32(N)


def kernel(action_features_actionframes, action_idxs_actionframes,
           examplars, examplars_variances):
    idx = action_idxs_actionframes.astype(jnp.int32)
    featT = action_features_actionframes.T
    return _sc_loss(featT, idx, examplars.T, examplars_variances.T)


# speedup vs baseline: 1.0300x; 1.0300x over previous
import functools

import jax
import jax.numpy as jnp
from jax import lax
from jax.experimental import pallas as pl
from jax.experimental.pallas import tpu as pltpu
from jax.experimental.pallas import tpu_sc as plsc

N, K, D = 16384, 100000, 64
NC, NS, LANES = 2, 16, 16
NW = NC * NS
ROWS_PER_W = N // NW
CHUNK = 128
NCHUNK = ROWS_PER_W // CHUNK
PCHUNK = 8192
NSTEP = -(-K // PCHUNK)
KP = NSTEP * PCHUNK


def _pack_body(eT_ref, vT_ref, ev_ref):
    i = lax.broadcasted_iota(jnp.int32, (D, D), 0)
    j = lax.broadcasted_iota(jnp.int32, (D, D), 1)
    eye = (i == j).astype(jnp.float32)
    dims = (((0,), (0,)), ((), ()))
    ev_ref[:, 0:D] = eT_ref[...].T
    ev_ref[:, D:2 * D] = lax.dot_general(
        vT_ref[...], eye, dims, preferred_element_type=jnp.float32)


def _pack_tables(exT, varT):
    return pl.pallas_call(
        _pack_body,
        grid=(NSTEP,),
        in_specs=[
            pl.BlockSpec((D, PCHUNK), lambda i: (0, i)),
            pl.BlockSpec((D, PCHUNK), lambda i: (0, i)),
        ],
        out_specs=pl.BlockSpec((PCHUNK, 2 * D), lambda i: (i, 0)),
        out_shape=jax.ShapeDtypeStruct((KP, 2 * D), jnp.float32),
    )(exT, varT)


def _sc_body(featT_hbm, idx_hbm, ev_hbm, out_hbm,
             idx_v, feat_v, ev_v, acc_v, sems):
    c = lax.axis_index("c")
    s = lax.axis_index("s")
    wid = s * NC + c
    base = wid * ROWS_PER_W

    pltpu.sync_copy(idx_hbm.at[pl.ds(base, ROWS_PER_W)], idx_v)
    pltpu.sync_copy(featT_hbm.at[:, pl.ds(base, ROWS_PER_W)], feat_v)

    def gather(j, buf):
        isl = idx_v.at[pl.ds(j * CHUNK, CHUNK)]
        return pltpu.async_copy(ev_hbm.at[isl], ev_v.at[buf], sems.at[buf])

    iota = lax.iota(jnp.int32, LANES)
    zero = jnp.zeros((LANES,), jnp.float32)
    accs = (zero, zero, zero, zero)
    pending = gather(0, 0)
    for j in range(NCHUNK):
        pending.wait()
        if j + 1 < NCHUNK:
            pending = gather(j + 1, (j + 1) % 2)
        buf = j % 2

        def row_body(r, accs):
            a0, a1, a2, a3 = accs
            p = jnp.full((LANES,), j * CHUNK + r, jnp.int32)
            f0 = plsc.load_gather(feat_v, [iota, p])
            e0 = ev_v[buf, r, pl.ds(0, LANES)]
            v0 = ev_v[buf, r, pl.ds(D, LANES)]
            a0 = a0 + jnp.abs(f0 - e0) / v0
            f1 = plsc.load_gather(feat_v, [iota + LANES, p])
            e1 = ev_v[buf, r, pl.ds(LANES, LANES)]
            v1 = ev_v[buf, r, pl.ds(D + LANES, LANES)]
            a1 = a1 + jnp.abs(f1 - e1) / v1
            f2 = plsc.load_gather(feat_v, [iota + 2 * LANES, p])
            e2 = ev_v[buf, r, pl.ds(2 * LANES, LANES)]
            v2 = ev_v[buf, r, pl.ds(D + 2 * LANES, LANES)]
            a2 = a2 + jnp.abs(f2 - e2) / v2
            f3 = plsc.load_gather(feat_v, [iota + 3 * LANES, p])
            e3 = ev_v[buf, r, pl.ds(3 * LANES, LANES)]
            v3 = ev_v[buf, r, pl.ds(D + 3 * LANES, LANES)]
            a3 = a3 + jnp.abs(f3 - e3) / v3
            return (a0, a1, a2, a3)

        accs = lax.fori_loop(0, CHUNK, row_body, accs)

    acc_v[...] = (accs[0] + accs[1]) + (accs[2] + accs[3])
    pltpu.sync_copy(acc_v, out_hbm.at[wid])


@jax.jit
def _sc_loss(featT, idx, exT, varT):
    ev2 = _pack_tables(exT, varT)
    mesh = plsc.VectorSubcoreMesh(core_axis_name="c", subcore_axis_name="s")
    partials = pl.kernel(
        _sc_body,
        mesh=mesh,
        out_type=jax.ShapeDtypeStruct((NW, LANES), jnp.float32),
        compiler_params=pltpu.CompilerParams(
            use_tc_tiling_on_sc=True, needs_layout_passes=False),
        scratch_types=[
            pltpu.VMEM((ROWS_PER_W,), jnp.int32),
            pltpu.VMEM((D, ROWS_PER_W), jnp.float32),
            pltpu.VMEM((2, CHUNK, 2 * D), jnp.float32),
            pltpu.VMEM((LANES,), jnp.float32),
            pltpu.SemaphoreType.DMA((2,)),
        ],
    )(featT, idx, ev2)
    return jnp.sum(partials) / jnp.float32(N)


def kernel(action_features_actionframes, action_idxs_actionframes,
           examplars, examplars_variances):
    idx = action_idxs_actionframes.astype(jnp.int32)
    featT = action_features_actionframes.T
    return _sc_loss(featT, idx, examplars.T, examplars_variances.T)
